# baseline (speedup 1.0000x reference)
import jax
import jax.numpy as jnp
from jax import lax
from jax.experimental import pallas as pl
from jax.experimental.pallas import tpu as pltpu
from jax.experimental.pallas import tpu_sc as plsc

N_CLS = 100000
CTX_DIM = 512
B = 1024
NC, NS = 2, 16
NW = NC * NS
BPW = B // NW
CHUNK = 8

D_ALL = 16 * CTX_DIM
D_CLS = 8 * CTX_DIM
D_GPH = 4 * CTX_DIM
D_SGL = 4 * CTX_DIM
ROW = D_ALL + D_CLS + D_GPH + D_SGL


def _sc_body(ci_hbm, gi_hbm, hi_hbm, sgl_hbm, all_hbm, cls_hbm, gph_hbm,
             out_hbm, all_v, ci_v, gi_v, hi_v, sgl_v, cls_v, gph_v,
             sem_s, sem_c, sem_g):
    wid = lax.axis_index("s") * NC + lax.axis_index("c")
    base = wid * BPW
    pltpu.sync_copy(all_hbm, all_v)
    for j in range(BPW // CHUNK):
        cb = base + j * CHUNK
        pltpu.sync_copy(ci_hbm.at[pl.ds(cb, CHUNK)], ci_v)
        pltpu.sync_copy(gi_hbm.at[pl.ds(cb, CHUNK)], gi_v)
        pltpu.sync_copy(hi_hbm.at[pl.ds(cb, CHUNK)], hi_v)
        cps = pltpu.async_copy(sgl_hbm.at[ci_v], sgl_v, sem_s)
        cpc = pltpu.async_copy(cls_hbm.at[gi_v], cls_v, sem_c)
        cpg = pltpu.async_copy(gph_hbm.at[hi_v], gph_v, sem_g)
        cps.wait()
        cpc.wait()
        cpg.wait()
        for e in range(CHUNK):
            r = cb + e
            pltpu.sync_copy(all_v, out_hbm.at[r, pl.ds(0, D_ALL)])
            pltpu.sync_copy(cls_v.at[e], out_hbm.at[r, pl.ds(D_ALL, D_CLS)])
            pltpu.sync_copy(gph_v.at[e],
                            out_hbm.at[r, pl.ds(D_ALL + D_CLS, D_GPH)])
            pltpu.sync_copy(sgl_v.at[e],
                            out_hbm.at[r, pl.ds(D_ALL + D_CLS + D_GPH, D_SGL)])


def kernel(cls_idx, cls_group_idx, graph_group_idx, ctx_single, ctx_all,
           ctx_cls, ctx_graph):
    mesh = plsc.VectorSubcoreMesh(core_axis_name="c", subcore_axis_name="s",
                                  num_cores=NC, num_subcores=NS)
    sgl2 = ctx_single.reshape(N_CLS, D_SGL)
    all1 = ctx_all.reshape(D_ALL)
    cls2 = ctx_cls.reshape(-1, D_CLS)
    gph2 = ctx_graph.reshape(-1, D_GPH)
    run = pl.kernel(
        _sc_body,
        out_type=jax.ShapeDtypeStruct((B, ROW), jnp.float32),
        mesh=mesh,
        scratch_types=[
            pltpu.VMEM((D_ALL,), jnp.float32),
            pltpu.VMEM((CHUNK,), jnp.int32),
            pltpu.VMEM((CHUNK,), jnp.int32),
            pltpu.VMEM((CHUNK,), jnp.int32),
            pltpu.VMEM((CHUNK, D_SGL), jnp.float32),
            pltpu.VMEM((CHUNK, D_CLS), jnp.float32),
            pltpu.VMEM((CHUNK, D_GPH), jnp.float32),
            pltpu.SemaphoreType.DMA,
            pltpu.SemaphoreType.DMA,
            pltpu.SemaphoreType.DMA,
        ],
    )
    out = run(cls_idx, cls_group_idx, graph_group_idx, sgl2, all1, cls2, gph2)
    return out.reshape(B, 32, CTX_DIM)


# speedup vs baseline: 1.0058x; 1.0058x over previous
import jax
import jax.numpy as jnp
from jax import lax
from jax.experimental import pallas as pl
from jax.experimental.pallas import tpu as pltpu
from jax.experimental.pallas import tpu_sc as plsc

N_CLS = 100000
CTX_DIM = 512
B = 1024
NC, NS = 2, 16
NW = NC * NS
BPW = B // NW
CHUNK = 8
NCH = BPW // CHUNK

D_ALL = 16 * CTX_DIM
D_CLS = 8 * CTX_DIM
D_GPH = 4 * CTX_DIM
D_SGL = 4 * CTX_DIM
ROW = D_ALL + D_CLS + D_GPH + D_SGL


def _sc_body(ci_hbm, gi_hbm, hi_hbm, sgl_hbm, all_hbm, cls_hbm, gph_hbm,
             out_hbm, all_v, ci_v, gi_v, hi_v, sgl_v, gph_v, cls_v,
             sem_s0, sem_s1, sem_c, sem_os0, sem_os1, sem_oc, sem_a):
    wid = lax.axis_index("s") * NC + lax.axis_index("c")
    base = wid * BPW
    sem_s = (sem_s0, sem_s1)
    sem_os = (sem_os0, sem_os1)

    pltpu.sync_copy(all_hbm, all_v)
    pltpu.sync_copy(ci_hbm.at[pl.ds(base, BPW)], ci_v)
    pltpu.sync_copy(gi_hbm.at[pl.ds(base, BPW)], gi_v)
    pltpu.sync_copy(hi_hbm.at[pl.ds(base, BPW)], hi_v)

    def fire_sg(j, bf):
        sl = pl.ds(j * CHUNK, CHUNK)
        return [
            pltpu.async_copy(sgl_hbm.at[ci_v.at[sl]], sgl_v.at[bf],
                             sem_s[bf]),
            pltpu.async_copy(gph_hbm.at[hi_v.at[sl]], gph_v.at[bf],
                             sem_s[bf]),
        ]

    def fire_cls(j):
        sl = pl.ds(j * CHUNK, CHUNK)
        return [pltpu.async_copy(cls_hbm.at[gi_v.at[sl]], cls_v, sem_c)]

    def fire_outs_sg(j, bf):
        cps = []
        for e in range(CHUNK):
            r = base + j * CHUNK + e
            cps.append(pltpu.async_copy(
                gph_v.at[bf, e],
                out_hbm.at[r, pl.ds(D_ALL + D_CLS, D_GPH)], sem_os[bf]))
            cps.append(pltpu.async_copy(
                sgl_v.at[bf, e],
                out_hbm.at[r, pl.ds(D_ALL + D_CLS + D_GPH, D_SGL)],
                sem_os[bf]))
        return cps

    def fire_outs_cls(j):
        return [pltpu.async_copy(cls_v.at[e],
                                 out_hbm.at[base + j * CHUNK + e,
                                            pl.ds(D_ALL, D_CLS)], sem_oc)
                for e in range(CHUNK)]

    def fire_outs_all(j):
        return [pltpu.async_copy(all_v,
                                 out_hbm.at[base + j * CHUNK + e,
                                            pl.ds(0, D_ALL)], sem_a)
                for e in range(CHUNK)]

    gd_sg = {0: fire_sg(0, 0)}
    gd_c = {0: fire_cls(0)}
    outs_sg = {0: [], 1: []}
    outs_c = []
    outs_a = []
    for j in range(NCH):
        bf = j % 2
        for d in gd_sg[j]:
            d.wait()
        for d in gd_c[j]:
            d.wait()
        prev_a, outs_a = outs_a, fire_outs_all(j)
        outs_sg[bf] = fire_outs_sg(j, bf)
        new_oc = fire_outs_cls(j)
        if j + 1 < NCH:
            nb = (j + 1) % 2
            for d in outs_sg[nb]:
                d.wait()
            outs_sg[nb] = []
            gd_sg[j + 1] = fire_sg(j + 1, nb)
            for d in outs_c:
                d.wait()
            for d in new_oc:
                d.wait()
            new_oc = []
            gd_c[j + 1] = fire_cls(j + 1)
            for d in prev_a:
                d.wait()
            prev_a = []
        outs_c = new_oc
    for d in outs_sg[0] + outs_sg[1] + outs_c + prev_a + outs_a:
        d.wait()


def kernel(cls_idx, cls_group_idx, graph_group_idx, ctx_single, ctx_all,
           ctx_cls, ctx_graph):
    mesh = plsc.VectorSubcoreMesh(core_axis_name="c", subcore_axis_name="s",
                                  num_cores=NC, num_subcores=NS)
    sgl2 = ctx_single.reshape(N_CLS, D_SGL)
    all1 = ctx_all.reshape(D_ALL)
    cls2 = ctx_cls.reshape(-1, D_CLS)
    gph2 = ctx_graph.reshape(-1, D_GPH)
    run = pl.kernel(
        _sc_body,
        out_type=jax.ShapeDtypeStruct((B, ROW), jnp.float32),
        mesh=mesh,
        scratch_types=[
            pltpu.VMEM((D_ALL,), jnp.float32),
            pltpu.VMEM((BPW,), jnp.int32),
            pltpu.VMEM((BPW,), jnp.int32),
            pltpu.VMEM((BPW,), jnp.int32),
            pltpu.VMEM((2, CHUNK, D_SGL), jnp.float32),
            pltpu.VMEM((2, CHUNK, D_GPH), jnp.float32),
            pltpu.VMEM((CHUNK, D_CLS), jnp.float32),
            pltpu.SemaphoreType.DMA,
            pltpu.SemaphoreType.DMA,
            pltpu.SemaphoreType.DMA,
            pltpu.SemaphoreType.DMA,
            pltpu.SemaphoreType.DMA,
            pltpu.SemaphoreType.DMA,
            pltpu.SemaphoreType.DMA,
        ],
    )
    out = run(cls_idx, cls_group_idx, graph_group_idx, sgl2, all1, cls2, gph2)
    return out.reshape(B, 32, CTX_DIM)


# speedup vs baseline: 9.3098x; 9.2565x over previous
import jax
import jax.numpy as jnp
from jax import lax
from jax.experimental import pallas as pl
from jax.experimental.pallas import tpu as pltpu
from jax.experimental.pallas import tpu_sc as plsc

N_CLS = 100000
CTX_DIM = 512
B = 1024
NC, NS = 2, 16
NW = NC * NS
BPW = B // NW
CHUNK = 8
NCH = BPW // CHUNK


def _sc_body(ci_hbm, gi_hbm, hi_hbm, sgl_hbm, all_hbm, cls_hbm, gph_hbm,
             out_hbm, all_v, ci_v, gi_v, hi_v, sgl_v, gph_v, cls_v,
             sem_g, sem_o, sem_a):
    wid = lax.axis_index("s") * NC + lax.axis_index("c")
    base = wid * BPW

    pltpu.sync_copy(all_hbm.at[0], all_v)
    pltpu.sync_copy(ci_hbm.at[pl.ds(base, BPW)], ci_v)
    pltpu.sync_copy(gi_hbm.at[pl.ds(base, BPW)], gi_v)
    pltpu.sync_copy(hi_hbm.at[pl.ds(base, BPW)], hi_v)

    for j in range(NCH):
        sl = pl.ds(j * CHUNK, CHUNK)
        g1 = pltpu.async_copy(sgl_hbm.at[ci_v.at[sl]], sgl_v, sem_g)
        g2 = pltpu.async_copy(cls_hbm.at[gi_v.at[sl]], cls_v, sem_g)
        g3 = pltpu.async_copy(gph_hbm.at[hi_v.at[sl]], gph_v, sem_g)
        g1.wait()
        g2.wait()
        g3.wait()
        outs = []
        for e in range(CHUNK):
            r = base + j * CHUNK + e
            outs.append(pltpu.async_copy(
                all_v, out_hbm.at[r, pl.ds(0, 16), :], sem_a))
            outs.append(pltpu.async_copy(
                cls_v.at[e], out_hbm.at[r, pl.ds(16, 8), :], sem_o))
            outs.append(pltpu.async_copy(
                gph_v.at[e], out_hbm.at[r, pl.ds(24, 4), :], sem_o))
            outs.append(pltpu.async_copy(
                sgl_v.at[e], out_hbm.at[r, pl.ds(28, 4), :], sem_o))
        for d in outs:
            d.wait()


def kernel(cls_idx, cls_group_idx, graph_group_idx, ctx_single, ctx_all,
           ctx_cls, ctx_graph):
    mesh = plsc.VectorSubcoreMesh(core_axis_name="c", subcore_axis_name="s",
                                  num_cores=NC, num_subcores=NS)
    run = pl.kernel(
        _sc_body,
        out_type=jax.ShapeDtypeStruct((B, 32, CTX_DIM), jnp.float32),
        mesh=mesh,
        compiler_params=pltpu.CompilerParams(use_tc_tiling_on_sc=True),
        scratch_types=[
            pltpu.VMEM((16, CTX_DIM), jnp.float32),
            pltpu.VMEM((BPW,), jnp.int32),
            pltpu.VMEM((BPW,), jnp.int32),
            pltpu.VMEM((BPW,), jnp.int32),
            pltpu.VMEM((CHUNK, 4, CTX_DIM), jnp.float32),
            pltpu.VMEM((CHUNK, 4, CTX_DIM), jnp.float32),
            pltpu.VMEM((CHUNK, 8, CTX_DIM), jnp.float32),
            pltpu.SemaphoreType.DMA,
            pltpu.SemaphoreType.DMA,
            pltpu.SemaphoreType.DMA,
        ],
    )
    return run(cls_idx, cls_group_idx, graph_group_idx, ctx_single, ctx_all,
               ctx_cls, ctx_graph)
